# baseline (speedup 1.0000x reference)
import jax
import jax.numpy as jnp
from jax import lax
from jax.experimental import pallas as pl
from jax.experimental.pallas import tpu as pltpu

B, L, D = 16, 2048, 1024
CH = 256
NCH = L // CH
MAXC = B * NCH
NBUF = 8


def _body(bid_r, st_r, rv_r, last_r, lenf_r, m_r,
          in_hbm, out_ref, buf, sem):
    m = m_r[0]

    def cp(j, slot):
        return pltpu.make_async_copy(
            in_hbm.at[bid_r[j], pl.ds(pl.multiple_of(st_r[j], CH), CH), :],
            buf.at[slot],
            sem.at[slot],
        )

    for t in range(NBUF - 1):
        @pl.when(t < m)
        def _():
            cp(t, t).start()

    def step(j, acc):
        slot = lax.rem(j, NBUF)
        jn = j + NBUF - 1

        @pl.when(jn < m)
        def _():
            cp(jn, lax.rem(jn, NBUF)).start()

        cp(j, slot).wait()
        rv = rv_r[j]

        def full_sum(_):
            return jnp.sum(buf[slot], axis=0)

        def masked_sum(_):
            row_id = lax.broadcasted_iota(jnp.int32, (CH, 1), 0)
            w = (row_id < rv).astype(jnp.float32)
            return jnp.sum(buf[slot] * w, axis=0)

        acc = acc + lax.cond(rv == CH, full_sum, masked_sum, 0)
        is_last = last_r[j] == 1

        @pl.when(is_last)
        def _():
            out_ref[bid_r[j], :] = acc / lenf_r[j]

        return jnp.where(is_last, 0.0, acc)

    lax.fori_loop(0, m, step, jnp.zeros((D,), jnp.float32))


def kernel(input, length):
    n = length.astype(jnp.int32)
    c = jnp.arange(NCH, dtype=jnp.int32)
    starts = c * CH
    valid = (starts[None, :] < n[:, None])
    bid_f = jnp.broadcast_to(
        jnp.arange(B, dtype=jnp.int32)[:, None], (B, NCH)).ravel()
    st_f = jnp.broadcast_to(starts[None, :], (B, NCH)).ravel()
    vflat = valid.ravel()
    m = vflat.sum(dtype=jnp.int32).reshape(1)
    (idx,) = jnp.nonzero(vflat, size=MAXC, fill_value=0)
    idx = idx.astype(jnp.int32)
    bid = bid_f[idx]
    st = st_f[idx]
    rv = jnp.minimum(n[bid] - st, CH)
    is_last = (st + CH >= n[bid]).astype(jnp.int32)
    lenf = n[bid].astype(jnp.float32)

    grid_spec = pltpu.PrefetchScalarGridSpec(
        num_scalar_prefetch=6,
        grid=(1,),
        in_specs=[pl.BlockSpec(memory_space=pl.ANY)],
        out_specs=pl.BlockSpec((B, D), lambda i, *_: (0, 0)),
        scratch_shapes=[
            pltpu.VMEM((NBUF, CH, D), jnp.float32),
            pltpu.SemaphoreType.DMA((NBUF,)),
        ],
    )
    return pl.pallas_call(
        _body,
        grid_spec=grid_spec,
        out_shape=jax.ShapeDtypeStruct((B, D), jnp.float32),
    )(bid, st, rv, is_last, lenf, m, input)


# speedup vs baseline: 1.2015x; 1.2015x over previous
import jax
import jax.numpy as jnp
from jax import lax
from jax.experimental import pallas as pl
from jax.experimental.pallas import tpu as pltpu

B, L, D = 16, 2048, 1024
CH = 512
NCH = L // CH
MAXC = B * NCH
NBUF = 4


def _body(bid_r, st_r, rv_r, last_r, lenf_r, m_r,
          in_hbm, out_ref, buf, sem):
    m = m_r[0]

    def cp(j, slot):
        return pltpu.make_async_copy(
            in_hbm.at[bid_r[j], pl.ds(pl.multiple_of(st_r[j], CH), CH), :],
            buf.at[slot],
            sem.at[slot],
        )

    for t in range(NBUF - 1):
        @pl.when(t < m)
        def _():
            cp(t, t).start()

    def step(j, acc):
        slot = lax.rem(j, NBUF)
        jn = j + NBUF - 1

        @pl.when(jn < m)
        def _():
            cp(jn, lax.rem(jn, NBUF)).start()

        cp(j, slot).wait()
        rv = rv_r[j]

        def full_sum(_):
            return jnp.sum(buf[slot], axis=0)

        def masked_sum(_):
            row_id = lax.broadcasted_iota(jnp.int32, (CH, 1), 0)
            w = (row_id < rv).astype(jnp.float32)
            return jnp.sum(buf[slot] * w, axis=0)

        acc = acc + lax.cond(rv == CH, full_sum, masked_sum, 0)
        is_last = last_r[j] == 1

        @pl.when(is_last)
        def _():
            out_ref[bid_r[j], :] = acc / lenf_r[j]

        return jnp.where(is_last, 0.0, acc)

    lax.fori_loop(0, m, step, jnp.zeros((D,), jnp.float32))


def kernel(input, length):
    n = length.astype(jnp.int32)
    c = jnp.arange(NCH, dtype=jnp.int32)
    starts = c * CH
    valid = (starts[None, :] < n[:, None])
    bid_f = jnp.broadcast_to(
        jnp.arange(B, dtype=jnp.int32)[:, None], (B, NCH)).ravel()
    st_f = jnp.broadcast_to(starts[None, :], (B, NCH)).ravel()
    vflat = valid.ravel()
    m = vflat.sum(dtype=jnp.int32).reshape(1)
    (idx,) = jnp.nonzero(vflat, size=MAXC, fill_value=0)
    idx = idx.astype(jnp.int32)
    bid = bid_f[idx]
    st = st_f[idx]
    rv = jnp.minimum(n[bid] - st, CH)
    is_last = (st + CH >= n[bid]).astype(jnp.int32)
    lenf = n[bid].astype(jnp.float32)

    grid_spec = pltpu.PrefetchScalarGridSpec(
        num_scalar_prefetch=6,
        grid=(1,),
        in_specs=[pl.BlockSpec(memory_space=pl.ANY)],
        out_specs=pl.BlockSpec((B, D), lambda i, *_: (0, 0)),
        scratch_shapes=[
            pltpu.VMEM((NBUF, CH, D), jnp.float32),
            pltpu.SemaphoreType.DMA((NBUF,)),
        ],
    )
    return pl.pallas_call(
        _body,
        grid_spec=grid_spec,
        out_shape=jax.ShapeDtypeStruct((B, D), jnp.float32),
    )(bid, st, rv, is_last, lenf, m, input)
